# baseline (speedup 1.0000x reference)
import functools

import jax
import jax.numpy as jnp
from jax import lax
from jax.experimental import pallas as pl
from jax.experimental.pallas import tpu as pltpu
from jax.experimental.pallas import tpu_sc as plsc

N = 10000
E = 320000
D = 128
L = 3

NPAD = 10240
NC = 2
NS = 16
NW = NC * NS
EW = E // NW
CH = 80
NBUF = 4
NCH = EW // CH
ACC_N = 10000
ROWS_T = ACC_N // NS

_mesh = plsc.VectorSubcoreMesh(core_axis_name="c", subcore_axis_name="s")


IDXP = 25


def _make_sc_agg(with_deg, nbuf, look, ch):
    nch = EW // ch

    def body(sdr_hbm, h_hbm, *refs):
        if with_deg:
            agg_hbm, deg_hbm, sd_v, rows_v, deg_v, sem_g, sem_s, acc_sh = refs
        else:
            agg_hbm, sd_v, rows_v, sem_g, sem_s, acc_sh = refs
        c = lax.axis_index("c")
        s = lax.axis_index("s")
        w = s * NC + c

        def _ldidx(p, _):
            pltpu.sync_copy(sdr_hbm.at[w, :, pl.ds(p * IDXP, IDXP), :],
                            sd_v.at[:, pl.ds(p * IDXP, IDXP), :])
            return 0
        lax.fori_loop(0, nch // IDXP, _ldidx, 0)

        zeros16 = jnp.zeros((16,), jnp.float32)
        ones16 = jnp.ones((16,), jnp.float32)

        def _zrow(i, _):
            rows_v[0, i // 8, pl.ds((i % 8) * 16, 16)] = zeros16
            return 0
        lax.fori_loop(0, ch * (D // 16), _zrow, 0)

        if with_deg:
            def _zdeg(i, _):
                deg_v[pl.ds(i * 16, 16)] = zeros16
                return 0
            lax.fori_loop(0, NPAD // 16, _zdeg, 0)

        for r in range(ROWS_T // ch):
            pltpu.sync_copy(rows_v.at[0],
                            acc_sh.at[pl.ds(s * ROWS_T + r * ch, ch), :])
        rem = ROWS_T % ch
        if rem:
            pltpu.sync_copy(
                rows_v.at[0, pl.ds(0, rem), :],
                acc_sh.at[pl.ds(s * ROWS_T + (ROWS_T // ch) * ch, rem), :])
        plsc.subcore_barrier()

        def _prime(i, _):
            pltpu.async_copy(h_hbm.at[sd_v.at[0, i]], rows_v.at[i],
                             sem_g.at[i])
            return 0
        lax.fori_loop(0, look, _prime, 0)

        def _chunk(k, _):
            b = k % nbuf
            kn = k + look
            bn = kn % nbuf

            pltpu.make_async_copy(h_hbm.at[sd_v.at[0, k]], rows_v.at[b],
                                  sem_g.at[b]).wait()
            pltpu.async_copy(rows_v.at[b], acc_sh.at[sd_v.at[1, k]],
                             sem_s.at[b], add=True)

            @pl.when(jnp.logical_and(k >= nbuf - look, kn < nch))
            def _():
                pltpu.make_async_copy(rows_v.at[bn],
                                      acc_sh.at[sd_v.at[1, kn - nbuf]],
                                      sem_s.at[bn]).wait()

            @pl.when(kn < nch)
            def _():
                pltpu.async_copy(h_hbm.at[sd_v.at[0, kn]], rows_v.at[bn],
                                 sem_g.at[bn])

            if with_deg:
                def _dsub(j, _):
                    dv = sd_v[1, k, pl.ds(j * 16, 16)]
                    plsc.addupdate_scatter(deg_v, [dv], ones16)
                    return 0
                lax.fori_loop(0, ch // 16, _dsub, 0)
            return 0

        lax.fori_loop(0, nch, _chunk, 0)
        for j in range(nbuf):
            kk = nch - nbuf + j
            pltpu.make_async_copy(rows_v.at[kk % nbuf],
                                  acc_sh.at[sd_v.at[1, kk]],
                                  sem_s.at[kk % nbuf]).wait()
        plsc.subcore_barrier()

        pltpu.sync_copy(acc_sh.at[pl.ds(s * ROWS_T, ROWS_T), :],
                        agg_hbm.at[c, pl.ds(s * ROWS_T, ROWS_T), :])
        if with_deg:
            pltpu.sync_copy(deg_v, deg_hbm.at[w])

    out_type = [jax.ShapeDtypeStruct((NC, NPAD, D), jnp.float32)]
    scratch = [
        pltpu.VMEM((2, nch, ch), jnp.int32),
        pltpu.VMEM((nbuf, ch, D), jnp.float32),
    ]
    if with_deg:
        out_type.append(jax.ShapeDtypeStruct((NW, NPAD), jnp.float32))
        scratch.append(pltpu.VMEM((NPAD,), jnp.float32))
    scratch += [
        pltpu.SemaphoreType.DMA((nbuf,)),
        pltpu.SemaphoreType.DMA((nbuf,)),
        pltpu.VMEM_SHARED((ACC_N, D), jnp.float32),
    ]
    return pl.kernel(
        body,
        out_type=tuple(out_type),
        mesh=_mesh,
        scratch_types=scratch,
        compiler_params=pltpu.CompilerParams(use_tc_tiling_on_sc=False,
                                             needs_layout_passes=False),
    )


CH_NODEG = 80
_sc_agg_nodeg = _make_sc_agg(False, 3, 2, CH_NODEG)


def _sc_deg_body(dstr_hbm, deg_hbm, dst_v, deg_v):
    c = lax.axis_index("c")
    s = lax.axis_index("s")
    w = s * NC + c

    def _ldidx(p, _):
        pltpu.sync_copy(dstr_hbm.at[w, pl.ds(p * 2000, 2000)],
                        dst_v.at[pl.ds(p * 2000, 2000)])
        return 0
    lax.fori_loop(0, EW // 2000, _ldidx, 0)

    zeros16 = jnp.zeros((16,), jnp.float32)
    ones16 = jnp.ones((16,), jnp.float32)

    def _zdeg(i, _):
        deg_v[pl.ds(i * 16, 16)] = zeros16
        return 0
    lax.fori_loop(0, NPAD // 16, _zdeg, 0)

    def _dsub(j, _):
        dv = dst_v[pl.ds(j * 16, 16)]
        plsc.addupdate_scatter(deg_v, [dv], ones16)
        return 0
    lax.fori_loop(0, EW // 16, _dsub, 0)

    pltpu.sync_copy(deg_v, deg_hbm.at[w])


_sc_deg = pl.kernel(
    _sc_deg_body,
    out_type=jax.ShapeDtypeStruct((NW, NPAD), jnp.float32),
    mesh=_mesh,
    scratch_types=[
        pltpu.VMEM((EW,), jnp.int32),
        pltpu.VMEM((NPAD,), jnp.float32),
    ],
    compiler_params=pltpu.CompilerParams(use_tc_tiling_on_sc=False,
                                         needs_layout_passes=False),
)


R = 512
GRID = NPAD // R


def _tc_layer_body(agg_ref, deg_ref, h_ref, wl_ref, bl_ref, wr_ref, out_ref):
    a = agg_ref[0] + agg_ref[1]
    dd = deg_ref[...]
    ones_c = jnp.full((NW, 1), 1.0, jnp.float32)
    deg_col = lax.dot_general(dd, ones_c, (((0,), (0,)), ((), ())),
                              preferred_element_type=jnp.float32)
    scale = 1.0 / jnp.maximum(deg_col, 1.0)
    mean = a * scale
    hn = (jnp.dot(mean, wl_ref[...], preferred_element_type=jnp.float32)
          + bl_ref[...]
          + jnp.dot(h_ref[...], wr_ref[...], preferred_element_type=jnp.float32))
    hn = jnp.maximum(hn, 0.0)
    row = (pl.program_id(0) * R
           + lax.broadcasted_iota(jnp.int32, (R, 1), 0))
    out_ref[...] = jnp.where(row < N, hn, 0.0)


_tc_layer = pl.pallas_call(
    _tc_layer_body,
    grid=(GRID,),
    in_specs=[
        pl.BlockSpec((NC, R, D), lambda i: (0, i, 0)),
        pl.BlockSpec((NW, R), lambda i: (0, i)),
        pl.BlockSpec((R, D), lambda i: (i, 0)),
        pl.BlockSpec((D, D), lambda i: (0, 0)),
        pl.BlockSpec((1, D), lambda i: (0, 0)),
        pl.BlockSpec((D, D), lambda i: (0, 0)),
    ],
    out_specs=pl.BlockSpec((R, D), lambda i: (i, 0)),
    out_shape=jax.ShapeDtypeStruct((NPAD, D), jnp.float32),
)


def _tc_last_body(agg_ref, deg_ref, h_ref, wl_ref, bl_ref, wr_ref,
                  x_ref, h1_ref, wf_ref, bf_ref, out_ref):
    a = agg_ref[0] + agg_ref[1]
    dd = deg_ref[...]
    ones_c = jnp.full((NW, 1), 1.0, jnp.float32)
    deg_col = lax.dot_general(dd, ones_c, (((0,), (0,)), ((), ())),
                              preferred_element_type=jnp.float32)
    scale = 1.0 / jnp.maximum(deg_col, 1.0)
    mean = a * scale
    h3 = (jnp.dot(mean, wl_ref[...], preferred_element_type=jnp.float32)
          + bl_ref[...]
          + jnp.dot(h_ref[...], wr_ref[...], preferred_element_type=jnp.float32))
    h3 = jnp.maximum(h3, 0.0)
    acc = jnp.dot(x_ref[...], wf_ref[pl.ds(0, D), :],
                  preferred_element_type=jnp.float32)
    acc += jnp.dot(h1_ref[...], wf_ref[pl.ds(D, D), :],
                   preferred_element_type=jnp.float32)
    acc += jnp.dot(h_ref[...], wf_ref[pl.ds(2 * D, D), :],
                   preferred_element_type=jnp.float32)
    acc += jnp.dot(h3, wf_ref[pl.ds(3 * D, D), :],
                   preferred_element_type=jnp.float32)
    out_ref[...] = acc + bf_ref[...]


_tc_last = pl.pallas_call(
    _tc_last_body,
    grid=(GRID,),
    in_specs=[
        pl.BlockSpec((NC, R, D), lambda i: (0, i, 0)),
        pl.BlockSpec((NW, R), lambda i: (0, i)),
        pl.BlockSpec((R, D), lambda i: (i, 0)),
        pl.BlockSpec((D, D), lambda i: (0, 0)),
        pl.BlockSpec((1, D), lambda i: (0, 0)),
        pl.BlockSpec((D, D), lambda i: (0, 0)),
        pl.BlockSpec((R, D), lambda i: (i, 0)),
        pl.BlockSpec((R, D), lambda i: (i, 0)),
        pl.BlockSpec(((1 + L) * D, D), lambda i: (0, 0)),
        pl.BlockSpec((1, D), lambda i: (0, 0)),
    ],
    out_specs=pl.BlockSpec((R, D), lambda i: (i, 0)),
    out_shape=jax.ShapeDtypeStruct((NPAD, D), jnp.float32),
)


def kernel(x, edge_index, edge_attr, Wl, bl, Wr, Wf, bf):
    del edge_attr
    sd_flat = jnp.stack([edge_index[0].reshape(NW, EW),
                         edge_index[1].reshape(NW, EW)], axis=1)
    sdr_nodeg = sd_flat.reshape(NW, 2, EW // CH_NODEG, CH_NODEG)
    dstr = edge_index[1].reshape(NW, EW)

    x_pad = jnp.zeros((NPAD, D), jnp.float32).at[:N].set(x)

    deg = _sc_deg(dstr)
    if isinstance(deg, (tuple, list)):
        deg = deg[0]

    agg = _sc_agg_nodeg(sdr_nodeg, x_pad)
    if isinstance(agg, (tuple, list)):
        agg = agg[0]
    h1 = _tc_layer(agg, deg, x_pad, Wl[0], bl[0][None], Wr[0])

    agg = _sc_agg_nodeg(sdr_nodeg, h1)
    if isinstance(agg, (tuple, list)):
        agg = agg[0]
    h2 = _tc_layer(agg, deg, h1, Wl[1], bl[1][None], Wr[1])

    agg = _sc_agg_nodeg(sdr_nodeg, h2)
    if isinstance(agg, (tuple, list)):
        agg = agg[0]
    out = _tc_last(agg, deg, h2, Wl[2], bl[2][None], Wr[2],
                   x_pad, h1, Wf, bf[None])
    return out[:N]


# speedup vs baseline: 1.0312x; 1.0312x over previous
import functools

import jax
import jax.numpy as jnp
from jax import lax
from jax.experimental import pallas as pl
from jax.experimental.pallas import tpu as pltpu
from jax.experimental.pallas import tpu_sc as plsc

N = 10000
E = 320000
D = 128
L = 3

NPAD = 10240
NC = 2
NS = 16
NW = NC * NS
EW = E // NW
CH = 80
NBUF = 4
NCH = EW // CH
ACC_N = 10000
ROWS_T = ACC_N // NS

_mesh = plsc.VectorSubcoreMesh(core_axis_name="c", subcore_axis_name="s")


IDXP = 25


def _make_sc_agg(with_deg, nbuf, look, ch):
    nch = EW // ch

    def body(sdr_hbm, h_hbm, *refs):
        if with_deg:
            (agg_hbm, deg_hbm, sd_v, rows_v, deg_v,
             sem_g, sem_s, sem_z, acc_sh) = refs
        else:
            agg_hbm, sd_v, rows_v, sem_g, sem_s, sem_z, acc_sh = refs
        c = lax.axis_index("c")
        s = lax.axis_index("s")
        w = s * NC + c

        def _ldidx(p, _):
            pltpu.sync_copy(sdr_hbm.at[w, :, pl.ds(p * IDXP, IDXP), :],
                            sd_v.at[:, pl.ds(p * IDXP, IDXP), :])
            return 0
        lax.fori_loop(0, nch // IDXP, _ldidx, 0)

        zeros16 = jnp.zeros((16,), jnp.float32)
        ones16 = jnp.ones((16,), jnp.float32)

        def _zrow(i, _):
            rows_v[0, i // 8, pl.ds((i % 8) * 16, 16)] = zeros16
            return 0
        lax.fori_loop(0, ch * (D // 16), _zrow, 0)

        if with_deg:
            def _zdeg(i, _):
                deg_v[pl.ds(i * 16, 16)] = zeros16
                return 0
            lax.fori_loop(0, NPAD // 16, _zdeg, 0)

        rem = ROWS_T % ch
        for r in range(ROWS_T // ch):
            pltpu.async_copy(rows_v.at[0],
                             acc_sh.at[pl.ds(s * ROWS_T + r * ch, ch), :],
                             sem_z)
        if rem:
            pltpu.async_copy(
                rows_v.at[0, pl.ds(0, rem), :],
                acc_sh.at[pl.ds(s * ROWS_T + (ROWS_T // ch) * ch, rem), :],
                sem_z)
        for r in range(ROWS_T // ch):
            pltpu.make_async_copy(
                rows_v.at[0],
                acc_sh.at[pl.ds(s * ROWS_T + r * ch, ch), :], sem_z).wait()
        if rem:
            pltpu.make_async_copy(
                rows_v.at[0, pl.ds(0, rem), :],
                acc_sh.at[pl.ds(s * ROWS_T + (ROWS_T // ch) * ch, rem), :],
                sem_z).wait()
        plsc.subcore_barrier()

        def _prime(i, _):
            pltpu.async_copy(h_hbm.at[sd_v.at[0, i]], rows_v.at[i],
                             sem_g.at[i])
            return 0
        lax.fori_loop(0, look, _prime, 0)

        def _chunk(k, _):
            b = k % nbuf
            kn = k + look
            bn = kn % nbuf

            @pl.when(jnp.logical_and(k >= nbuf - look, kn < nch))
            def _():
                pltpu.make_async_copy(rows_v.at[bn],
                                      acc_sh.at[sd_v.at[1, kn - nbuf]],
                                      sem_s.at[bn]).wait()

            @pl.when(kn < nch)
            def _():
                pltpu.async_copy(h_hbm.at[sd_v.at[0, kn]], rows_v.at[bn],
                                 sem_g.at[bn])

            pltpu.make_async_copy(h_hbm.at[sd_v.at[0, k]], rows_v.at[b],
                                  sem_g.at[b]).wait()
            pltpu.async_copy(rows_v.at[b], acc_sh.at[sd_v.at[1, k]],
                             sem_s.at[b], add=True)

            if with_deg:
                def _dsub(j, _):
                    dv = sd_v[1, k, pl.ds(j * 16, 16)]
                    plsc.addupdate_scatter(deg_v, [dv], ones16)
                    return 0
                lax.fori_loop(0, ch // 16, _dsub, 0)
            return 0

        lax.fori_loop(0, nch, _chunk, 0)
        for j in range(nbuf):
            kk = nch - nbuf + j
            pltpu.make_async_copy(rows_v.at[kk % nbuf],
                                  acc_sh.at[sd_v.at[1, kk]],
                                  sem_s.at[kk % nbuf]).wait()
        plsc.subcore_barrier()

        pltpu.sync_copy(acc_sh.at[pl.ds(s * ROWS_T, ROWS_T), :],
                        agg_hbm.at[c, pl.ds(s * ROWS_T, ROWS_T), :])
        if with_deg:
            pltpu.sync_copy(deg_v, deg_hbm.at[w])

    out_type = [jax.ShapeDtypeStruct((NC, NPAD, D), jnp.float32)]
    scratch = [
        pltpu.VMEM((2, nch, ch), jnp.int32),
        pltpu.VMEM((nbuf, ch, D), jnp.float32),
    ]
    if with_deg:
        out_type.append(jax.ShapeDtypeStruct((NW, NPAD), jnp.float32))
        scratch.append(pltpu.VMEM((NPAD,), jnp.float32))
    scratch += [
        pltpu.SemaphoreType.DMA((nbuf,)),
        pltpu.SemaphoreType.DMA((nbuf,)),
        pltpu.SemaphoreType.DMA,
        pltpu.VMEM_SHARED((ACC_N, D), jnp.float32),
    ]
    return pl.kernel(
        body,
        out_type=tuple(out_type),
        mesh=_mesh,
        scratch_types=scratch,
        compiler_params=pltpu.CompilerParams(use_tc_tiling_on_sc=False,
                                             needs_layout_passes=False),
    )


CH_NODEG = 80
_sc_agg_nodeg = _make_sc_agg(False, 3, 2, CH_NODEG)


def _sc_deg_body(dstr_hbm, deg_hbm, dst_v, deg_v):
    c = lax.axis_index("c")
    s = lax.axis_index("s")
    w = s * NC + c

    def _ldidx(p, _):
        pltpu.sync_copy(dstr_hbm.at[w, pl.ds(p * 2000, 2000)],
                        dst_v.at[pl.ds(p * 2000, 2000)])
        return 0
    lax.fori_loop(0, EW // 2000, _ldidx, 0)

    zeros16 = jnp.zeros((16,), jnp.float32)
    ones16 = jnp.ones((16,), jnp.float32)

    def _zdeg(i, _):
        deg_v[pl.ds(i * 16, 16)] = zeros16
        return 0
    lax.fori_loop(0, NPAD // 16, _zdeg, 0)

    def _dsub(j, _):
        dv = dst_v[pl.ds(j * 16, 16)]
        plsc.addupdate_scatter(deg_v, [dv], ones16)
        return 0
    lax.fori_loop(0, EW // 16, _dsub, 0)

    pltpu.sync_copy(deg_v, deg_hbm.at[w])


_sc_deg = pl.kernel(
    _sc_deg_body,
    out_type=jax.ShapeDtypeStruct((NW, NPAD), jnp.float32),
    mesh=_mesh,
    scratch_types=[
        pltpu.VMEM((EW,), jnp.int32),
        pltpu.VMEM((NPAD,), jnp.float32),
    ],
    compiler_params=pltpu.CompilerParams(use_tc_tiling_on_sc=False,
                                         needs_layout_passes=False),
)


R = 512
GRID = NPAD // R


def _tc_layer_body(agg_ref, deg_ref, h_ref, wl_ref, bl_ref, wr_ref, out_ref):
    a = agg_ref[0] + agg_ref[1]
    dd = deg_ref[...]
    ones_c = jnp.full((NW, 1), 1.0, jnp.float32)
    deg_col = lax.dot_general(dd, ones_c, (((0,), (0,)), ((), ())),
                              preferred_element_type=jnp.float32)
    scale = 1.0 / jnp.maximum(deg_col, 1.0)
    mean = a * scale
    hn = (jnp.dot(mean, wl_ref[...], preferred_element_type=jnp.float32)
          + bl_ref[...]
          + jnp.dot(h_ref[...], wr_ref[...], preferred_element_type=jnp.float32))
    hn = jnp.maximum(hn, 0.0)
    row = (pl.program_id(0) * R
           + lax.broadcasted_iota(jnp.int32, (R, 1), 0))
    out_ref[...] = jnp.where(row < N, hn, 0.0)


_tc_layer = pl.pallas_call(
    _tc_layer_body,
    grid=(GRID,),
    in_specs=[
        pl.BlockSpec((NC, R, D), lambda i: (0, i, 0)),
        pl.BlockSpec((NW, R), lambda i: (0, i)),
        pl.BlockSpec((R, D), lambda i: (i, 0)),
        pl.BlockSpec((D, D), lambda i: (0, 0)),
        pl.BlockSpec((1, D), lambda i: (0, 0)),
        pl.BlockSpec((D, D), lambda i: (0, 0)),
    ],
    out_specs=pl.BlockSpec((R, D), lambda i: (i, 0)),
    out_shape=jax.ShapeDtypeStruct((NPAD, D), jnp.float32),
)


def _tc_last_body(agg_ref, deg_ref, h_ref, wl_ref, bl_ref, wr_ref,
                  x_ref, h1_ref, wf_ref, bf_ref, out_ref):
    a = agg_ref[0] + agg_ref[1]
    dd = deg_ref[...]
    ones_c = jnp.full((NW, 1), 1.0, jnp.float32)
    deg_col = lax.dot_general(dd, ones_c, (((0,), (0,)), ((), ())),
                              preferred_element_type=jnp.float32)
    scale = 1.0 / jnp.maximum(deg_col, 1.0)
    mean = a * scale
    h3 = (jnp.dot(mean, wl_ref[...], preferred_element_type=jnp.float32)
          + bl_ref[...]
          + jnp.dot(h_ref[...], wr_ref[...], preferred_element_type=jnp.float32))
    h3 = jnp.maximum(h3, 0.0)
    acc = jnp.dot(x_ref[...], wf_ref[pl.ds(0, D), :],
                  preferred_element_type=jnp.float32)
    acc += jnp.dot(h1_ref[...], wf_ref[pl.ds(D, D), :],
                   preferred_element_type=jnp.float32)
    acc += jnp.dot(h_ref[...], wf_ref[pl.ds(2 * D, D), :],
                   preferred_element_type=jnp.float32)
    acc += jnp.dot(h3, wf_ref[pl.ds(3 * D, D), :],
                   preferred_element_type=jnp.float32)
    out_ref[...] = acc + bf_ref[...]


_tc_last = pl.pallas_call(
    _tc_last_body,
    grid=(GRID,),
    in_specs=[
        pl.BlockSpec((NC, R, D), lambda i: (0, i, 0)),
        pl.BlockSpec((NW, R), lambda i: (0, i)),
        pl.BlockSpec((R, D), lambda i: (i, 0)),
        pl.BlockSpec((D, D), lambda i: (0, 0)),
        pl.BlockSpec((1, D), lambda i: (0, 0)),
        pl.BlockSpec((D, D), lambda i: (0, 0)),
        pl.BlockSpec((R, D), lambda i: (i, 0)),
        pl.BlockSpec((R, D), lambda i: (i, 0)),
        pl.BlockSpec(((1 + L) * D, D), lambda i: (0, 0)),
        pl.BlockSpec((1, D), lambda i: (0, 0)),
    ],
    out_specs=pl.BlockSpec((R, D), lambda i: (i, 0)),
    out_shape=jax.ShapeDtypeStruct((NPAD, D), jnp.float32),
)


def kernel(x, edge_index, edge_attr, Wl, bl, Wr, Wf, bf):
    del edge_attr
    sd_flat = jnp.stack([edge_index[0].reshape(NW, EW),
                         edge_index[1].reshape(NW, EW)], axis=1)
    sdr_nodeg = sd_flat.reshape(NW, 2, EW // CH_NODEG, CH_NODEG)
    dstr = edge_index[1].reshape(NW, EW)

    x_pad = jnp.zeros((NPAD, D), jnp.float32).at[:N].set(x)

    deg = _sc_deg(dstr)
    if isinstance(deg, (tuple, list)):
        deg = deg[0]

    agg = _sc_agg_nodeg(sdr_nodeg, x_pad)
    if isinstance(agg, (tuple, list)):
        agg = agg[0]
    h1 = _tc_layer(agg, deg, x_pad, Wl[0], bl[0][None], Wr[0])

    agg = _sc_agg_nodeg(sdr_nodeg, h1)
    if isinstance(agg, (tuple, list)):
        agg = agg[0]
    h2 = _tc_layer(agg, deg, h1, Wl[1], bl[1][None], Wr[1])

    agg = _sc_agg_nodeg(sdr_nodeg, h2)
    if isinstance(agg, (tuple, list)):
        agg = agg[0]
    out = _tc_last(agg, deg, h2, Wl[2], bl[2][None], Wr[2],
                   x_pad, h1, Wf, bf[None])
    return out[:N]
